# baseline (device time: 308553 ns/iter reference)
import jax
import jax.numpy as jnp
from jax import lax
from jax.experimental import pallas as pl
from jax.experimental.pallas import tpu as pltpu

N = 4096
D = 1024
B = 128
PADC = B
NOUT = N // B
NBS = NOUT + 1


def kernel(x, dest):
    dest = dest.astype(jnp.int32)
    my_y = lax.axis_index("y")
    is0 = my_y == 0

    x_bf = x.astype(jnp.bfloat16)

    isz = dest == 0
    czc = jnp.cumsum(isz.astype(jnp.int32))
    coc = jnp.cumsum(1 - isz.astype(jnp.int32))
    cz = czc[N - 1]
    inv = jnp.where(isz, czc - 1, cz + coc - 1)
    perm = jnp.zeros((N,), jnp.int32).at[inv].set(
        jnp.arange(N, dtype=jnp.int32)
    )

    s = jnp.where(is0, N - cz, cz)
    sh = jnp.where(is0, 0, (N - s) % 8)
    p = jnp.arange(N + B, dtype=jnp.int32)
    src_idx = jnp.clip(jnp.where(is0, cz + p, p - sh), 0, N - 1)
    gidx = jnp.concatenate([perm, perm[src_idx]])
    g = x_bf[gidx]

    def body(cz_ref, g_ref, out_ref, c_ref, send_sems, recv_sems):
        my_x = lax.axis_index("x")
        my_yv = lax.axis_index("y")
        my_z = lax.axis_index("z")
        partner = (my_x, 1 - my_yv, my_z)
        i0 = my_yv == 0

        czv = cz_ref[0]
        sv = jnp.where(i0, N - czv, czv)
        kv = N - sv
        sh_snd = jnp.where(i0, 0, (N - sv) % 8)
        dst_base = jnp.where(i0, PADC, PADC + (N - sv) - sh_snd)
        dst_base = pl.multiple_of(dst_base, 8)
        nb_snd = (sh_snd + sv + B - 1) // B
        sh_rcv = jnp.where(i0, kv % 8, 0)
        rcv_base = jnp.where(i0, PADC + kv - sh_rcv, PADC)
        rcv_base = pl.multiple_of(rcv_base, 8)
        nb_rcv = (sh_rcv + sv + B - 1) // B

        barrier = pltpu.get_barrier_semaphore()
        pl.semaphore_signal(
            barrier, inc=1, device_id=partner,
            device_id_type=pl.DeviceIdType.MESH,
        )
        pl.semaphore_wait(barrier, 1)

        for j in range(NBS):
            @pl.when(j < nb_snd)
            def _():
                rdma = pltpu.make_async_remote_copy(
                    src_ref=g_ref.at[pl.ds(N + j * B, B)],
                    dst_ref=c_ref.at[pl.ds(dst_base + j * B, B)],
                    send_sem=send_sems.at[j],
                    recv_sem=recv_sems.at[j],
                    device_id=partner,
                    device_id_type=pl.DeviceIdType.MESH,
                )
                rdma.start()

        lo = jnp.where(i0, 0, sv)
        hi = jnp.where(i0, kv, N)
        for j in range(NOUT):
            @pl.when((j * B >= lo) & ((j + 1) * B <= hi))
            def _():
                out_ref[j * B:(j + 1) * B, :] = g_ref[j * B:(j + 1) * B, :]

        for j in range(NBS):
            @pl.when(j < nb_rcv)
            def _():
                rdma = pltpu.make_async_remote_copy(
                    src_ref=g_ref.at[pl.ds(0, B)],
                    dst_ref=c_ref.at[pl.ds(rcv_base + j * B, B)],
                    send_sem=send_sems.at[j],
                    recv_sem=recv_sems.at[j],
                    device_id=partner,
                    device_id_type=pl.DeviceIdType.MESH,
                )
                rdma.wait_recv()

        for j in range(NOUT):
            @pl.when((j * B < lo) | ((j + 1) * B > hi))
            def _():
                row = j * B + lax.broadcasted_iota(jnp.int32, (B, 1), 0)
                take_sorted = (row >= lo) & (row < hi)
                out_ref[j * B:(j + 1) * B, :] = jnp.where(
                    take_sorted,
                    g_ref[j * B:(j + 1) * B, :],
                    c_ref[PADC + j * B:PADC + (j + 1) * B, :],
                )

        for j in range(NBS):
            @pl.when(j < nb_snd)
            def _():
                rdma = pltpu.make_async_remote_copy(
                    src_ref=g_ref.at[pl.ds(0, B)],
                    dst_ref=c_ref.at[pl.ds(0, B)],
                    send_sem=send_sems.at[j],
                    recv_sem=recv_sems.at[j],
                    device_id=partner,
                    device_id_type=pl.DeviceIdType.MESH,
                )
                rdma.wait_send()

    return pl.pallas_call(
        body,
        out_shape=jax.ShapeDtypeStruct((N, D), jnp.bfloat16),
        in_specs=[
            pl.BlockSpec(memory_space=pltpu.SMEM),
            pl.BlockSpec(memory_space=pltpu.VMEM),
        ],
        out_specs=pl.BlockSpec(memory_space=pltpu.VMEM),
        scratch_shapes=[
            pltpu.VMEM((N + 2 * B, D), jnp.bfloat16),
            pltpu.SemaphoreType.DMA((NBS,)),
            pltpu.SemaphoreType.DMA((NBS,)),
        ],
        compiler_params=pltpu.CompilerParams(collective_id=0),
    )(cz.reshape((1,)), g)


# device time: 118981 ns/iter; 2.5933x vs baseline; 2.5933x over previous
import jax
import jax.numpy as jnp
from jax import lax
from jax.experimental import pallas as pl
from jax.experimental.pallas import tpu as pltpu

N = 4096
D = 1024
B = 128
PADC = B
NOUT = N // B
CAP = 3 * N // 4
NBS = CAP // B


def kernel(x, dest):
    dest = dest.astype(jnp.int32)
    my_y = lax.axis_index("y")
    is0 = my_y == 0

    x_bf = x.astype(jnp.bfloat16)

    isz = dest == 0
    czc = jnp.cumsum(isz.astype(jnp.int32))
    coc = jnp.cumsum(1 - isz.astype(jnp.int32))
    cz = czc[N - 1]
    inv = jnp.where(isz, czc - 1, cz + coc - 1)
    perm = jnp.zeros((N,), jnp.int32).at[inv].set(
        jnp.arange(N, dtype=jnp.int32)
    )

    s = jnp.where(is0, N - cz, cz)
    sh = jnp.where(is0, 0, (N - s) % 8)
    p = jnp.arange(CAP, dtype=jnp.int32)
    src_idx = jnp.clip(jnp.where(is0, cz + p, p - sh), 0, N - 1)
    sorted_x = x_bf[perm]
    buf_send = sorted_x[src_idx]

    def body(cz_ref, g_ref, send_ref, out_ref, c_ref, send_sems, recv_sems):
        my_x = lax.axis_index("x")
        my_yv = lax.axis_index("y")
        my_z = lax.axis_index("z")
        partner = (my_x, 1 - my_yv, my_z)
        i0 = my_yv == 0

        czv = cz_ref[0]
        sv = jnp.where(i0, N - czv, czv)
        kv = N - sv
        sh_snd = jnp.where(i0, 0, (N - sv) % 8)
        dst_base = jnp.where(i0, PADC, PADC + (N - sv) - sh_snd)
        dst_base = pl.multiple_of(dst_base, 8)
        nb_snd = (sh_snd + sv + B - 1) // B
        sh_rcv = jnp.where(i0, kv % 8, 0)
        rcv_base = jnp.where(i0, PADC + kv - sh_rcv, PADC)
        rcv_base = pl.multiple_of(rcv_base, 8)
        nb_rcv = (sh_rcv + sv + B - 1) // B

        barrier = pltpu.get_barrier_semaphore()
        pl.semaphore_signal(
            barrier, inc=1, device_id=partner,
            device_id_type=pl.DeviceIdType.MESH,
        )
        pl.semaphore_wait(barrier, 1)

        for j in range(NBS):
            @pl.when(j < nb_snd)
            def _():
                rdma = pltpu.make_async_remote_copy(
                    src_ref=send_ref.at[pl.ds(j * B, B)],
                    dst_ref=c_ref.at[pl.ds(dst_base + j * B, B)],
                    send_sem=send_sems.at[j],
                    recv_sem=recv_sems.at[j],
                    device_id=partner,
                    device_id_type=pl.DeviceIdType.MESH,
                )
                rdma.start()

        lo = jnp.where(i0, 0, sv)
        hi = jnp.where(i0, kv, N)
        for j in range(NOUT):
            @pl.when((j * B >= lo) & ((j + 1) * B <= hi))
            def _():
                out_ref[j * B:(j + 1) * B, :] = g_ref[j * B:(j + 1) * B, :]

        for j in range(NBS):
            @pl.when(j < nb_rcv)
            def _():
                rdma = pltpu.make_async_remote_copy(
                    src_ref=g_ref.at[pl.ds(0, B)],
                    dst_ref=c_ref.at[pl.ds(rcv_base + j * B, B)],
                    send_sem=send_sems.at[j],
                    recv_sem=recv_sems.at[j],
                    device_id=partner,
                    device_id_type=pl.DeviceIdType.MESH,
                )
                rdma.wait_recv()

        for j in range(NOUT):
            @pl.when((j * B < lo) | ((j + 1) * B > hi))
            def _():
                row = j * B + lax.broadcasted_iota(jnp.int32, (B, 1), 0)
                take_sorted = (row >= lo) & (row < hi)
                out_ref[j * B:(j + 1) * B, :] = jnp.where(
                    take_sorted,
                    g_ref[j * B:(j + 1) * B, :],
                    c_ref[PADC + j * B:PADC + (j + 1) * B, :],
                )

        for j in range(NBS):
            @pl.when(j < nb_snd)
            def _():
                rdma = pltpu.make_async_remote_copy(
                    src_ref=g_ref.at[pl.ds(0, B)],
                    dst_ref=c_ref.at[pl.ds(0, B)],
                    send_sem=send_sems.at[j],
                    recv_sem=recv_sems.at[j],
                    device_id=partner,
                    device_id_type=pl.DeviceIdType.MESH,
                )
                rdma.wait_send()

    return pl.pallas_call(
        body,
        out_shape=jax.ShapeDtypeStruct((N, D), jnp.bfloat16),
        in_specs=[
            pl.BlockSpec(memory_space=pltpu.SMEM),
            pl.BlockSpec(memory_space=pltpu.VMEM),
            pl.BlockSpec(memory_space=pltpu.VMEM),
        ],
        out_specs=pl.BlockSpec(memory_space=pltpu.VMEM),
        scratch_shapes=[
            pltpu.VMEM((N + 2 * B, D), jnp.bfloat16),
            pltpu.SemaphoreType.DMA((NBS,)),
            pltpu.SemaphoreType.DMA((NBS,)),
        ],
        compiler_params=pltpu.CompilerParams(collective_id=0),
    )(cz.reshape((1,)), sorted_x, buf_send)


# device time: 108533 ns/iter; 2.8429x vs baseline; 1.0963x over previous
import jax
import jax.numpy as jnp
from jax import lax
from jax.experimental import pallas as pl
from jax.experimental.pallas import tpu as pltpu

N = 4096
D = 1024
B = 128
PADC = B
NOUT = N // B
CAP = 3 * N // 4
NBS = CAP // B


def kernel(x, dest):
    dest = dest.astype(jnp.int32)
    my_y = lax.axis_index("y")
    is0 = my_y == 0

    x_bf = x.astype(jnp.bfloat16)

    isz = dest == 0
    t = jnp.arange(N, dtype=jnp.int32)
    czc = jnp.cumsum(isz.astype(jnp.int32))
    coc = t + 1 - czc
    cz = czc[N - 1]
    pz = jnp.sum(czc[None, :] <= t[:, None], axis=1, dtype=jnp.int32)
    po = jnp.sum(coc[None, :] <= t[:, None], axis=1, dtype=jnp.int32)
    po_shift = lax.dynamic_slice(
        jnp.concatenate([jnp.zeros((N,), jnp.int32), po]), (N - cz,), (N,)
    )
    perm = jnp.where(t < cz, pz, po_shift)
    sorted_x = x_bf[perm]

    s = jnp.where(is0, N - cz, cz)
    sh = jnp.where(is0, 0, (N - s) % 8)
    padded = jnp.concatenate(
        [
            jnp.zeros((8, D), jnp.bfloat16),
            sorted_x,
            jnp.zeros((CAP, D), jnp.bfloat16),
        ]
    )
    start = jnp.where(is0, 8 + cz, 8 - sh)
    buf_send = lax.dynamic_slice(padded, (start, 0), (CAP, D))

    def body(cz_ref, g_ref, send_ref, out_ref, c_ref, send_sems, recv_sems):
        my_x = lax.axis_index("x")
        my_yv = lax.axis_index("y")
        my_z = lax.axis_index("z")
        partner = (my_x, 1 - my_yv, my_z)
        i0 = my_yv == 0

        czv = cz_ref[0]
        sv = jnp.where(i0, N - czv, czv)
        kv = N - sv
        sh_snd = jnp.where(i0, 0, (N - sv) % 8)
        dst_base = jnp.where(i0, PADC, PADC + (N - sv) - sh_snd)
        dst_base = pl.multiple_of(dst_base, 8)
        nb_snd = (sh_snd + sv + B - 1) // B
        sh_rcv = jnp.where(i0, kv % 8, 0)
        rcv_base = jnp.where(i0, PADC + kv - sh_rcv, PADC)
        rcv_base = pl.multiple_of(rcv_base, 8)
        nb_rcv = (sh_rcv + sv + B - 1) // B

        barrier = pltpu.get_barrier_semaphore()
        pl.semaphore_signal(
            barrier, inc=1, device_id=partner,
            device_id_type=pl.DeviceIdType.MESH,
        )
        pl.semaphore_wait(barrier, 1)

        for j in range(NBS):
            @pl.when(j < nb_snd)
            def _():
                rdma = pltpu.make_async_remote_copy(
                    src_ref=send_ref.at[pl.ds(j * B, B)],
                    dst_ref=c_ref.at[pl.ds(dst_base + j * B, B)],
                    send_sem=send_sems.at[j],
                    recv_sem=recv_sems.at[j],
                    device_id=partner,
                    device_id_type=pl.DeviceIdType.MESH,
                )
                rdma.start()

        lo = jnp.where(i0, 0, sv)
        hi = jnp.where(i0, kv, N)
        for j in range(NOUT):
            @pl.when((j * B >= lo) & ((j + 1) * B <= hi))
            def _():
                out_ref[j * B:(j + 1) * B, :] = g_ref[j * B:(j + 1) * B, :]

        for j in range(NBS):
            @pl.when(j < nb_rcv)
            def _():
                rdma = pltpu.make_async_remote_copy(
                    src_ref=g_ref.at[pl.ds(0, B)],
                    dst_ref=c_ref.at[pl.ds(rcv_base + j * B, B)],
                    send_sem=send_sems.at[j],
                    recv_sem=recv_sems.at[j],
                    device_id=partner,
                    device_id_type=pl.DeviceIdType.MESH,
                )
                rdma.wait_recv()

        for j in range(NOUT):
            @pl.when((j * B < lo) | ((j + 1) * B > hi))
            def _():
                row = j * B + lax.broadcasted_iota(jnp.int32, (B, 1), 0)
                take_sorted = (row >= lo) & (row < hi)
                out_ref[j * B:(j + 1) * B, :] = jnp.where(
                    take_sorted,
                    g_ref[j * B:(j + 1) * B, :],
                    c_ref[PADC + j * B:PADC + (j + 1) * B, :],
                )

        for j in range(NBS):
            @pl.when(j < nb_snd)
            def _():
                rdma = pltpu.make_async_remote_copy(
                    src_ref=g_ref.at[pl.ds(0, B)],
                    dst_ref=c_ref.at[pl.ds(0, B)],
                    send_sem=send_sems.at[j],
                    recv_sem=recv_sems.at[j],
                    device_id=partner,
                    device_id_type=pl.DeviceIdType.MESH,
                )
                rdma.wait_send()

    return pl.pallas_call(
        body,
        out_shape=jax.ShapeDtypeStruct((N, D), jnp.bfloat16),
        in_specs=[
            pl.BlockSpec(memory_space=pltpu.SMEM),
            pl.BlockSpec(memory_space=pltpu.VMEM),
            pl.BlockSpec(memory_space=pltpu.VMEM),
        ],
        out_specs=pl.BlockSpec(memory_space=pltpu.VMEM),
        scratch_shapes=[
            pltpu.VMEM((N + 2 * B, D), jnp.bfloat16),
            pltpu.SemaphoreType.DMA((NBS,)),
            pltpu.SemaphoreType.DMA((NBS,)),
        ],
        compiler_params=pltpu.CompilerParams(collective_id=0),
    )(cz.reshape((1,)), sorted_x, buf_send)


# device time: 86121 ns/iter; 3.5828x vs baseline; 1.2602x over previous
import jax
import jax.numpy as jnp
from jax import lax
from jax.experimental import pallas as pl
from jax.experimental.pallas import tpu as pltpu

N = 4096
D = 1024
B = 128
PADC = B
NOUT = N // B
CAP = 3 * N // 4
NBS = CAP // B


def kernel(x, dest):
    dest = dest.astype(jnp.int32)
    my_y = lax.axis_index("y")
    is0 = my_y == 0

    x_bf = x.astype(jnp.bfloat16)

    isz = dest == 0
    t = jnp.arange(N, dtype=jnp.int32)
    czc = jnp.cumsum(isz.astype(jnp.int32))
    coc = t + 1 - czc
    cz = czc[N - 1]
    inv = jnp.where(isz, czc - 1, cz + coc - 1)
    sorted_x = (
        jnp.zeros((N, D), jnp.bfloat16)
        .at[inv]
        .set(x_bf, unique_indices=True)
    )

    s = jnp.where(is0, N - cz, cz)
    sh = jnp.where(is0, 0, (N - s) % 8)
    padded = jnp.concatenate(
        [
            jnp.zeros((8, D), jnp.bfloat16),
            sorted_x,
            jnp.zeros((CAP, D), jnp.bfloat16),
        ]
    )
    start = jnp.where(is0, 8 + cz, 8 - sh)
    buf_send = lax.dynamic_slice(padded, (start, 0), (CAP, D))

    def body(cz_ref, g_ref, send_ref, out_ref, c_ref, send_sems, recv_sems):
        my_x = lax.axis_index("x")
        my_yv = lax.axis_index("y")
        my_z = lax.axis_index("z")
        partner = (my_x, 1 - my_yv, my_z)
        i0 = my_yv == 0

        czv = cz_ref[0]
        sv = jnp.where(i0, N - czv, czv)
        kv = N - sv
        sh_snd = jnp.where(i0, 0, (N - sv) % 8)
        dst_base = jnp.where(i0, PADC, PADC + (N - sv) - sh_snd)
        dst_base = pl.multiple_of(dst_base, 8)
        nb_snd = (sh_snd + sv + B - 1) // B
        sh_rcv = jnp.where(i0, kv % 8, 0)
        rcv_base = jnp.where(i0, PADC + kv - sh_rcv, PADC)
        rcv_base = pl.multiple_of(rcv_base, 8)
        nb_rcv = (sh_rcv + sv + B - 1) // B

        barrier = pltpu.get_barrier_semaphore()
        pl.semaphore_signal(
            barrier, inc=1, device_id=partner,
            device_id_type=pl.DeviceIdType.MESH,
        )
        pl.semaphore_wait(barrier, 1)

        for j in range(NBS):
            @pl.when(j < nb_snd)
            def _():
                rdma = pltpu.make_async_remote_copy(
                    src_ref=send_ref.at[pl.ds(j * B, B)],
                    dst_ref=c_ref.at[pl.ds(dst_base + j * B, B)],
                    send_sem=send_sems.at[j],
                    recv_sem=recv_sems.at[j],
                    device_id=partner,
                    device_id_type=pl.DeviceIdType.MESH,
                )
                rdma.start()

        lo = jnp.where(i0, 0, sv)
        hi = jnp.where(i0, kv, N)
        for j in range(NOUT):
            @pl.when((j * B >= lo) & ((j + 1) * B <= hi))
            def _():
                out_ref[j * B:(j + 1) * B, :] = g_ref[j * B:(j + 1) * B, :]

        for j in range(NBS):
            @pl.when(j < nb_rcv)
            def _():
                rdma = pltpu.make_async_remote_copy(
                    src_ref=g_ref.at[pl.ds(0, B)],
                    dst_ref=c_ref.at[pl.ds(rcv_base + j * B, B)],
                    send_sem=send_sems.at[j],
                    recv_sem=recv_sems.at[j],
                    device_id=partner,
                    device_id_type=pl.DeviceIdType.MESH,
                )
                rdma.wait_recv()

        for j in range(NOUT):
            @pl.when((j * B < lo) | ((j + 1) * B > hi))
            def _():
                row = j * B + lax.broadcasted_iota(jnp.int32, (B, 1), 0)
                take_sorted = (row >= lo) & (row < hi)
                out_ref[j * B:(j + 1) * B, :] = jnp.where(
                    take_sorted,
                    g_ref[j * B:(j + 1) * B, :],
                    c_ref[PADC + j * B:PADC + (j + 1) * B, :],
                )

        for j in range(NBS):
            @pl.when(j < nb_snd)
            def _():
                rdma = pltpu.make_async_remote_copy(
                    src_ref=g_ref.at[pl.ds(0, B)],
                    dst_ref=c_ref.at[pl.ds(0, B)],
                    send_sem=send_sems.at[j],
                    recv_sem=recv_sems.at[j],
                    device_id=partner,
                    device_id_type=pl.DeviceIdType.MESH,
                )
                rdma.wait_send()

    return pl.pallas_call(
        body,
        out_shape=jax.ShapeDtypeStruct((N, D), jnp.bfloat16),
        in_specs=[
            pl.BlockSpec(memory_space=pltpu.SMEM),
            pl.BlockSpec(memory_space=pltpu.VMEM),
            pl.BlockSpec(memory_space=pltpu.VMEM),
        ],
        out_specs=pl.BlockSpec(memory_space=pltpu.VMEM),
        scratch_shapes=[
            pltpu.VMEM((N + 2 * B, D), jnp.bfloat16),
            pltpu.SemaphoreType.DMA((NBS,)),
            pltpu.SemaphoreType.DMA((NBS,)),
        ],
        compiler_params=pltpu.CompilerParams(collective_id=0),
    )(cz.reshape((1,)), sorted_x, buf_send)
